# baseline (speedup 1.0000x reference)
import functools

import jax
import jax.numpy as jnp
from jax import lax
from jax.experimental import pallas as pl
from jax.experimental.pallas import tpu as pltpu
from jax.experimental.pallas import tpu_sc as plsc

B = 4096
F = 20
D = 64
H = 512
A = 4

NC = 2
NS = 16
NW = NC * NS
BPW = B // NW
SUB = 32
NSUB = BPW // SUB
RPS = SUB * F
GCH = 128
NG = RPS // GCH
LANES = 16
DV = D // LANES


def _pool_body(idx_hbm, table_hbm, out_hbm, idx_v, rows_v, pooled_v, sem):
    wid = lax.axis_index("s") * NC + lax.axis_index("c")
    base_row = wid * BPW

    pltpu.sync_copy(idx_hbm.at[pl.ds(base_row * F, BPW * F)], idx_v)

    def fire(sc, buf):
        handles = []
        for g in range(NG):
            h = pltpu.async_copy(
                table_hbm.at[idx_v.at[pl.ds(sc * RPS + g * GCH, GCH)]],
                rows_v.at[buf].at[pl.ds(g * GCH, GCH)],
                sem,
            )
            handles.append(h)
        return handles

    def pool_and_store(sc, buf):
        rows = rows_v.at[buf]

        def body_b(b, _):
            r0 = b * F
            for c in range(DV):
                acc = rows[r0, pl.ds(c * LANES, LANES)]
                for f in range(1, F):
                    acc = acc + rows[r0 + f, pl.ds(c * LANES, LANES)]
                pooled_v[b, pl.ds(c * LANES, LANES)] = acc
            return 0

        lax.fori_loop(0, SUB, body_b, 0)
        pltpu.sync_copy(pooled_v, out_hbm.at[pl.ds(base_row + sc * SUB, SUB)])

    pending = fire(0, 0)
    for sc in range(NSUB):
        buf = sc % 2
        for h in pending:
            h.wait()
        if sc + 1 < NSUB:
            pending = fire(sc + 1, (sc + 1) % 2)
        pool_and_store(sc, buf)


@functools.partial(jax.jit, static_argnames=())
def _gather_pool(idx_flat, table):
    mesh = plsc.VectorSubcoreMesh(core_axis_name="c", subcore_axis_name="s")
    kern = functools.partial(
        pl.kernel,
        out_type=jax.ShapeDtypeStruct((B, D), jnp.float32),
        mesh=mesh,
        scratch_types=[
            pltpu.VMEM((BPW * F,), jnp.int32),
            pltpu.VMEM((2, RPS, D), jnp.float32),
            pltpu.VMEM((SUB, D), jnp.float32),
            pltpu.SemaphoreType.DMA,
        ],
        compiler_params=pltpu.CompilerParams(use_tc_tiling_on_sc=False),
    )(_pool_body)
    return kern(idx_flat, table)


def _mlp_body(p_ref, w1_ref, b1_ref, w2_ref, b2_ref, y_ref):
    h = jnp.dot(p_ref[...], w1_ref[...], preferred_element_type=jnp.float32)
    h = jnp.maximum(h + b1_ref[...], 0.0)
    y_ref[...] = jnp.dot(h, w2_ref[...], preferred_element_type=jnp.float32) + b2_ref[...]


MLP_BLK = 1024


def _mlp(pooled, W1, b1, W2, b2):
    return pl.pallas_call(
        _mlp_body,
        grid=(B // MLP_BLK,),
        in_specs=[
            pl.BlockSpec((MLP_BLK, D), lambda i: (i, 0)),
            pl.BlockSpec((D, H), lambda i: (0, 0)),
            pl.BlockSpec((1, H), lambda i: (0, 0)),
            pl.BlockSpec((H, A), lambda i: (0, 0)),
            pl.BlockSpec((1, A), lambda i: (0, 0)),
        ],
        out_specs=pl.BlockSpec((MLP_BLK, A), lambda i: (i, 0)),
        out_shape=jax.ShapeDtypeStruct((B, A), jnp.float32),
    )(pooled, W1, b1, W2, b2)


def kernel(indices, table, W1, b1, W2, b2):
    idx_flat = indices.reshape(-1).astype(jnp.int32)
    pooled = _gather_pool(idx_flat, table)
    return _mlp(pooled, W1, b1.reshape(1, H), W2, b2.reshape(1, A))
